# baseline (speedup 1.0000x reference)
import jax
import jax.numpy as jnp
from jax import lax
from jax.experimental import pallas as pl
from jax.experimental.pallas import tpu as pltpu
from jax.experimental.pallas import tpu_sc as plsc

B, S, D = 16, 2048, 512
CH = 128
NV = D // 16
NW = 32


def _scalar_at(vec_ref, i):
    return vec_ref[pl.ds(i, 16)][0]


def _sc_body(x_hbm, n_hbm, part_hbm, nvec_ref, buf_ref, part_ref, sem):
    c = lax.axis_index("c")
    s = lax.axis_index("s")
    w = s * 2 + c

    pltpu.sync_copy(n_hbm, nvec_ref.at[pl.ds(0, 16)])

    def tot_body(j, t):
        return t + _scalar_at(nvec_ref, j)
    T = lax.fori_loop(0, B, tot_body, jnp.int32(0))

    lo = w * T // NW
    hi = (w + 1) * T // NW

    zero = jnp.zeros((16,), jnp.float32)

    def zero_body(r, _):
        for j in range(NV):
            part_ref[r, pl.ds(j * 16, 16)] = zero
        return 0
    lax.fori_loop(0, B, zero_body, 0)

    def batch_body(i, C):
        n_i = _scalar_at(nvec_ref, i)
        a = jnp.maximum(lo, C)
        b = jnp.minimum(hi, C + n_i)

        @pl.when(b > a)
        def _():
            r0 = a - C
            r1 = b - C
            c0 = r0 // CH
            c1 = (r1 + CH - 1) // CH

            def chunk_body(cc, accs):
                pltpu.async_copy(
                    x_hbm.at[i, pl.ds(cc * CH, CH)], buf_ref, sem).wait()
                lo_r = jnp.maximum(r0 - cc * CH, 0)
                hi_r = jnp.minimum(r1 - cc * CH, CH)

                def row_body(r, acc):
                    return tuple(acc[j] + buf_ref[r, pl.ds(j * 16, 16)]
                                 for j in range(NV))

                return lax.fori_loop(lo_r, hi_r, row_body, accs)

            accs0 = tuple(jnp.zeros((16,), jnp.float32) for _ in range(NV))
            accs = lax.fori_loop(c0, c1, chunk_body, accs0)
            for j in range(NV):
                part_ref[i, pl.ds(j * 16, 16)] = accs[j]

        return C + n_i

    lax.fori_loop(0, B, batch_body, jnp.int32(0))

    pltpu.sync_copy(part_ref, part_hbm.at[w])


def _combine_body(part_ref, nf_ref, out_ref):
    out_ref[...] = jnp.sum(part_ref[...], axis=0) / nf_ref[...]


def kernel(x, N):
    mesh = plsc.VectorSubcoreMesh(core_axis_name="c", subcore_axis_name="s")
    sc = pl.kernel(
        _sc_body,
        out_type=jax.ShapeDtypeStruct((NW, B, D), jnp.float32),
        mesh=mesh,
        scratch_types=[
            pltpu.VMEM((32,), jnp.int32),
            pltpu.VMEM((CH, D), jnp.float32),
            pltpu.VMEM((B, D), jnp.float32),
            pltpu.SemaphoreType.DMA,
        ],
    )
    partials = sc(x, N)
    nf = N.astype(jnp.float32).reshape(B, 1)
    return pl.pallas_call(
        _combine_body,
        out_shape=jax.ShapeDtypeStruct((B, D), jnp.float32),
    )(partials, nf)


# speedup vs baseline: 1.1591x; 1.1591x over previous
import jax
import jax.numpy as jnp
from jax import lax
from jax.experimental import pallas as pl
from jax.experimental.pallas import tpu as pltpu
from jax.experimental.pallas import tpu_sc as plsc

B, S, D = 16, 2048, 512
CH = 64
NV = D // 16
HNV = NV // 2
NW = 32


def _scalar_at(vec_ref, i):
    return vec_ref[pl.ds(i, 16)][0]


def _sc_body(x_hbm, n_hbm, part_hbm, nvec_ref, buf0_ref, buf1_ref, part_ref,
             sem0, sem1):
    c = lax.axis_index("c")
    s = lax.axis_index("s")
    w = s * 2 + c

    pltpu.sync_copy(n_hbm, nvec_ref.at[pl.ds(0, 16)])

    def tot_body(j, t):
        return t + _scalar_at(nvec_ref, j)
    T = lax.fori_loop(0, B, tot_body, jnp.int32(0))

    lo = w * T // NW
    hi = (w + 1) * T // NW

    zero = jnp.zeros((16,), jnp.float32)

    def zero_body(r, _):
        for j in range(NV):
            part_ref[r, pl.ds(j * 16, 16)] = zero
        return 0
    lax.fori_loop(0, B, zero_body, 0)

    bufs = (buf0_ref, buf1_ref)
    sems = (sem0, sem1)

    def batch_body(i, C):
        n_i = _scalar_at(nvec_ref, i)
        a = jnp.maximum(lo, C)
        b = jnp.minimum(hi, C + n_i)

        @pl.when(b > a)
        def _():
            r0 = a - C
            r1 = b - C
            c0 = r0 // CH
            c1 = (r1 + CH - 1) // CH

            def start(chunk, bi):
                @pl.when(chunk < c1)
                def _():
                    pltpu.async_copy(
                        x_hbm.at[i, pl.ds(chunk * CH, CH)], bufs[bi],
                        sems[bi])

            def wait(bi):
                pltpu.make_async_copy(
                    x_hbm.at[i, pl.ds(0, CH)], bufs[bi], sems[bi]).wait()

            start(c0, 0)
            start(c0 + 1, 1)

            def accum_chunk(chunk, bi, accs):
                lo_r = jnp.maximum(r0 - chunk * CH, 0)
                hi_r = jnp.minimum(r1 - chunk * CH, CH)
                buf = bufs[bi]

                @pl.when(chunk < c1)
                def _():
                    wait(bi)

                accs_lo, accs_hi = accs[:HNV], accs[HNV:]

                def row_lo(r, a):
                    return tuple(a[j] + buf[r, pl.ds(j * 16, 16)]
                                 for j in range(HNV))

                def row_hi(r, a):
                    return tuple(a[j] + buf[r, pl.ds((HNV + j) * 16, 16)]
                                 for j in range(HNV))

                accs_lo = lax.fori_loop(lo_r, hi_r, row_lo, accs_lo)
                accs_hi = lax.fori_loop(lo_r, hi_r, row_hi, accs_hi)
                start(chunk + 2, bi)
                return accs_lo + accs_hi

            def pair_body(it, accs):
                chunk = c0 + 2 * it
                accs = accum_chunk(chunk, 0, accs)
                accs = accum_chunk(chunk + 1, 1, accs)
                return accs

            accs0 = tuple(jnp.zeros((16,), jnp.float32) for _ in range(NV))
            npairs = (c1 - c0 + 1) // 2
            accs = lax.fori_loop(0, npairs, pair_body, accs0)
            for j in range(NV):
                part_ref[i, pl.ds(j * 16, 16)] = accs[j]

        return C + n_i

    lax.fori_loop(0, B, batch_body, jnp.int32(0))

    pltpu.sync_copy(part_ref, part_hbm.at[w])


def _combine_body(part_ref, nf_ref, out_ref):
    out_ref[...] = jnp.sum(part_ref[...], axis=0) / nf_ref[...]


def kernel(x, N):
    mesh = plsc.VectorSubcoreMesh(core_axis_name="c", subcore_axis_name="s")
    sc = pl.kernel(
        _sc_body,
        out_type=jax.ShapeDtypeStruct((NW, B, D), jnp.float32),
        mesh=mesh,
        scratch_types=[
            pltpu.VMEM((32,), jnp.int32),
            pltpu.VMEM((CH, D), jnp.float32),
            pltpu.VMEM((CH, D), jnp.float32),
            pltpu.VMEM((B, D), jnp.float32),
            pltpu.SemaphoreType.DMA,
            pltpu.SemaphoreType.DMA,
        ],
    )
    partials = sc(x, N)
    nf = N.astype(jnp.float32).reshape(B, 1)
    return pl.pallas_call(
        _combine_body,
        out_shape=jax.ShapeDtypeStruct((B, D), jnp.float32),
    )(partials, nf)
